# Initial kernel scaffold; baseline (speedup 1.0000x reference)
import jax
import jax.numpy as jnp
from jax.experimental import pallas as pl


def kernel(token_ids, W_word, W_pos, W_type):
    raise NotImplementedError("write your pallas kernel here")


# speedup vs baseline: 1.6967x; 1.6967x over previous
import functools

import jax
import jax.numpy as jnp
from jax import lax
from jax.experimental import pallas as pl
from jax.experimental.pallas import tpu as pltpu
from jax.experimental.pallas import tpu_sc as plsc

VOCAB = 1000000
HID = 64
MAXPOS = 512
TYPEV = 2
B = 1024
S = 200

NUM_CORES = 2
NUM_SUBCORES = 16
NW = NUM_CORES * NUM_SUBCORES
N_TOK = B * S
PER_W = N_TOK // NW
CHUNK = 128
N_CHUNK = PER_W // CHUNK


def _word_gather_body(tok_hbm, table_hbm, out_hbm, idx_v, rows_v, sem):
    wid = lax.axis_index("s") * NUM_CORES + lax.axis_index("c")
    base0 = wid * PER_W

    def step(j, _):
        base = base0 + j * CHUNK
        pltpu.sync_copy(tok_hbm.at[pl.ds(base, CHUNK)], idx_v)
        pltpu.async_copy(table_hbm.at[idx_v], rows_v, sem).wait()
        pltpu.sync_copy(rows_v, out_hbm.at[pl.ds(base, CHUNK)])
        return _

    lax.fori_loop(0, N_CHUNK, step, None)


@jax.jit
def _word_gather(tok_flat, w_word):
    mesh = plsc.VectorSubcoreMesh(core_axis_name="c", subcore_axis_name="s")
    return pl.kernel(
        _word_gather_body,
        out_type=jax.ShapeDtypeStruct((N_TOK, HID), jnp.float32),
        mesh=mesh,
        scratch_types=[
            pltpu.VMEM((CHUNK,), jnp.int32),
            pltpu.VMEM((CHUNK, HID), jnp.float32),
            pltpu.SemaphoreType.DMA,
        ],
        compiler_params=pltpu.CompilerParams(use_tc_tiling_on_sc=False),
    )(tok_flat, w_word)


ROWS = S * HID // 128
BB = 16


def _bcast_body(pos_ref, type_ref, pos_out, type_out):
    pos = pos_ref[...]
    t0 = type_ref[0:1, :]
    trow = jnp.concatenate([t0, t0], axis=1)
    for b in range(BB):
        pos_out[pl.ds(b * ROWS, ROWS), :] = pos
    type_out[...] = jnp.broadcast_to(trow, (BB * ROWS, 128))


@jax.jit
def _broadcasts(w_pos, w_type):
    pos2d = w_pos[:S].reshape(ROWS, 128)
    out_shape = jax.ShapeDtypeStruct((B * ROWS, 128), jnp.float32)
    pos_flat, type_flat = pl.pallas_call(
        _bcast_body,
        grid=(B // BB,),
        in_specs=[
            pl.BlockSpec((ROWS, 128), lambda i: (0, 0)),
            pl.BlockSpec((TYPEV, HID), lambda i: (0, 0)),
        ],
        out_specs=[
            pl.BlockSpec((BB * ROWS, 128), lambda i: (i, 0)),
            pl.BlockSpec((BB * ROWS, 128), lambda i: (i, 0)),
        ],
        out_shape=[out_shape, out_shape],
    )(pos2d, w_type)
    return pos_flat.reshape(B, S, HID), type_flat.reshape(B, S, HID)


def kernel(token_ids, W_word, W_pos, W_type):
    tok_flat = token_ids.reshape(-1).astype(jnp.int32)
    word = _word_gather(tok_flat, W_word).reshape(B, S, HID)
    pos, typ = _broadcasts(W_pos, W_type)
    return (word, pos, typ)


# speedup vs baseline: 2.4747x; 1.4586x over previous
import jax
import jax.numpy as jnp
from jax import lax
from jax.experimental import pallas as pl
from jax.experimental.pallas import tpu as pltpu
from jax.experimental.pallas import tpu_sc as plsc

VOCAB = 1000000
HID = 64
MAXPOS = 512
TYPEV = 2
B = 1024
S = 200

NUM_CORES = 2
NUM_SUBCORES = 16
NW = NUM_CORES * NUM_SUBCORES
N_TOK = B * S
PER_W = N_TOK // NW
CHUNK = 640
N_CHUNK = PER_W // CHUNK


def _word_gather_body(tok_hbm, table_hbm, out_hbm, idx_v, rows_v, sem):
    wid = lax.axis_index("s") * NUM_CORES + lax.axis_index("c")
    base0 = wid * PER_W

    def chunk_step(c, _):
        base = base0 + c * CHUNK
        pltpu.sync_copy(tok_hbm.at[pl.ds(base, CHUNK)], idx_v)

        def fire(g, carry):
            vec = idx_v[pl.ds(g * 16, 16)]
            for j in range(16):
                r = vec[j]
                pltpu.async_copy(
                    table_hbm.at[pl.ds(r, 1)],
                    rows_v.at[pl.ds(g * 16 + j, 1)],
                    sem,
                )
            return carry

        lax.fori_loop(0, CHUNK // 16, fire, None)
        pltpu.make_async_copy(table_hbm.at[pl.ds(0, CHUNK)], rows_v, sem).wait()
        pltpu.sync_copy(rows_v, out_hbm.at[pl.ds(base, CHUNK)])
        return _

    lax.fori_loop(0, N_CHUNK, chunk_step, None)


@jax.jit
def _word_gather(tok_flat, w_word):
    mesh = plsc.VectorSubcoreMesh(core_axis_name="c", subcore_axis_name="s")
    return pl.kernel(
        _word_gather_body,
        out_type=jax.ShapeDtypeStruct((N_TOK, HID), jnp.float32),
        mesh=mesh,
        scratch_types=[
            pltpu.VMEM((CHUNK,), jnp.int32),
            pltpu.VMEM((CHUNK, HID), jnp.float32),
            pltpu.SemaphoreType.DMA,
        ],
        compiler_params=pltpu.CompilerParams(use_tc_tiling_on_sc=True),
    )(tok_flat, w_word)


BB = 32


def _bcast_body(pos_ref, type_ref, pos_out, type_out):
    pos_out[...] = jnp.broadcast_to(pos_ref[...][None], (BB, S, HID))
    type_out[...] = jnp.broadcast_to(type_ref[0:1, :][None], (BB, S, HID))


@jax.jit
def _broadcasts(w_pos, w_type):
    out_shape = jax.ShapeDtypeStruct((B, S, HID), jnp.float32)
    return pl.pallas_call(
        _bcast_body,
        grid=(B // BB,),
        in_specs=[
            pl.BlockSpec((S, HID), lambda i: (0, 0)),
            pl.BlockSpec((TYPEV, HID), lambda i: (0, 0)),
        ],
        out_specs=[
            pl.BlockSpec((BB, S, HID), lambda i: (i, 0, 0)),
            pl.BlockSpec((BB, S, HID), lambda i: (i, 0, 0)),
        ],
        out_shape=[out_shape, out_shape],
    )(w_pos[:S], w_type)


def kernel(token_ids, W_word, W_pos, W_type):
    tok_flat = token_ids.reshape(-1).astype(jnp.int32)
    word = _word_gather(tok_flat, W_word).reshape(B, S, HID)
    pos, typ = _broadcasts(W_pos, W_type)
    return (word, pos, typ)
